# Initial kernel scaffold; baseline (speedup 1.0000x reference)
import jax
import jax.numpy as jnp
from jax.experimental import pallas as pl


def kernel(x, edge_index, emb_W, emb_b, W, a, out_W, out_b):
    raise NotImplementedError("write your pallas kernel here")


# speedup vs baseline: 4.8224x; 4.8224x over previous
import functools

import jax
import jax.numpy as jnp
from jax import lax
from jax.experimental import pallas as pl
from jax.experimental.pallas import tpu as pltpu
from jax.experimental.pallas import tpu_sc as plsc

N = 10000
E = 320000
D = 128
H = 128
C = 128

NC = 2
NS = 16
LANES = 16
NW = NC * NS
EPT = E // NW
CH = 80
NCHUNK = EPT // CH
NGRP = CH // LANES
NP = 10240
RPT = NP // NS
RCP = 8
NEG = -9e15


def _dense_in_body(x_ref, ew_ref, eb_ref, w_ref, a2_ref, wh_ref, al_ref):
    h = jnp.dot(x_ref[...], ew_ref[...], preferred_element_type=jnp.float32)
    h = h + eb_ref[...][None, :]
    wh = jnp.dot(h, w_ref[...], preferred_element_type=jnp.float32)
    wh_ref[...] = wh
    al_ref[...] = jnp.dot(wh, a2_ref[...], preferred_element_type=jnp.float32)


def _dense_in(x, emb_W, emb_b, W, a2):
    blk = 2000
    grid = (N // blk,)
    return pl.pallas_call(
        _dense_in_body,
        grid=grid,
        in_specs=[
            pl.BlockSpec((blk, D), lambda i: (i, 0)),
            pl.BlockSpec((D, H), lambda i: (0, 0)),
            pl.BlockSpec((H,), lambda i: (0,)),
            pl.BlockSpec((H, H), lambda i: (0, 0)),
            pl.BlockSpec((H, 2), lambda i: (0, 0)),
        ],
        out_specs=[
            pl.BlockSpec((blk, H), lambda i: (i, 0)),
            pl.BlockSpec((blk, 2), lambda i: (i, 0)),
        ],
        out_shape=[
            jax.ShapeDtypeStruct((N, H), jnp.float32),
            jax.ShapeDtypeStruct((N, 2), jnp.float32),
        ],
    )(x, emb_W, emb_b, W, a2)


def _edge_body(src_hbm, dst_hbm, asrc_hbm, adst_hbm, wh_hbm,
               part_hbm, z_hbm, m_hbm,
               idx_s, idx_d, avals, bvals, masked_v, rows_v, zbuf_v,
               vec_v, stage_v, asrc_sh, adst_sh, acc_sh, stage_sh, sem):
    cid = lax.axis_index("c")
    sid = lax.axis_index("s")
    gid = cid * NS + sid
    ebase = gid * EPT

    @pl.when(sid == 0)
    def _():
        pltpu.sync_copy(asrc_hbm, asrc_sh)
        pltpu.sync_copy(adst_hbm, adst_sh)

    zeros16 = jnp.zeros((LANES,), jnp.float32)
    for r in range(RCP):
        for f in range(H // LANES):
            zbuf_v[r, pl.ds(f * LANES, LANES)] = zeros16

    def _zero_step(j, carry):
        pltpu.sync_copy(zbuf_v, acc_sh.at[pl.ds(sid * RPT + j * RCP, RCP)])
        return carry
    lax.fori_loop(0, RPT // RCP, _zero_step, 0)
    plsc.subcore_barrier()

    def _p1_step(c, mx):
        base = ebase + c * CH
        pltpu.sync_copy(src_hbm.at[pl.ds(base, CH)], idx_s)
        pltpu.sync_copy(dst_hbm.at[pl.ds(base, CH)], idx_d)
        pltpu.async_copy(asrc_sh.at[idx_s], avals, sem).wait()
        pltpu.async_copy(adst_sh.at[idx_d], bvals, sem).wait()
        for g in range(NGRP):
            sl = pl.ds(g * LANES, LANES)
            s = avals[sl] + bvals[sl]
            masked = jnp.where(s > 0.0, s, jnp.float32(NEG))
            masked_v[c, sl] = masked
            mx = jnp.maximum(mx, masked)
        return mx

    mx0 = jnp.full((LANES,), jnp.float32(NEG))
    mx = lax.fori_loop(0, NCHUNK, _p1_step, mx0)

    vec_v[...] = mx
    pltpu.sync_copy(vec_v, stage_sh.at[sid])
    plsc.subcore_barrier()
    pltpu.sync_copy(stage_sh, stage_v)
    mloc = jnp.full((LANES,), jnp.float32(NEG))
    for t in range(NS):
        mloc = jnp.maximum(mloc, stage_v[t, :])
    m_sc = jnp.max(mloc)

    def _p2_step(c, z):
        base = ebase + c * CH
        pltpu.sync_copy(src_hbm.at[pl.ds(base, CH)], idx_s)
        pltpu.sync_copy(dst_hbm.at[pl.ds(base, CH)], idx_d)
        ws = []
        for g in range(NGRP):
            mk = masked_v[c, pl.ds(g * LANES, LANES)]
            w = jnp.exp(mk - m_sc)
            z = z + w
            ws.append(w)
        pltpu.async_copy(wh_hbm.at[idx_s], rows_v, sem).wait()
        for g in range(NGRP):
            for k in range(LANES):
                wb = lax.broadcast(ws[g][k], (LANES,))
                r = g * LANES + k
                for f in range(H // LANES):
                    sl = pl.ds(f * LANES, LANES)
                    rows_v[r, sl] = rows_v[r, sl] * wb
        pltpu.sync_copy(rows_v, acc_sh.at[idx_d], add=True)
        return z

    z0 = jnp.zeros((LANES,), jnp.float32)
    z = lax.fori_loop(0, NCHUNK, _p2_step, z0)

    vec_v[...] = z
    pltpu.sync_copy(vec_v, z_hbm.at[cid, sid])

    @pl.when(sid == 0)
    def _():
        vec_v[...] = lax.broadcast(m_sc, (LANES,))
        pltpu.sync_copy(vec_v, m_hbm.at[cid])

    plsc.subcore_barrier()
    pltpu.sync_copy(acc_sh.at[pl.ds(sid * RPT, RPT)],
                    part_hbm.at[cid, pl.ds(sid * RPT, RPT)])


def _edge_kernel(src1, dst1, asrc, adst, wh):
    mesh = plsc.VectorSubcoreMesh(core_axis_name="c", subcore_axis_name="s")
    fn = pl.kernel(
        _edge_body,
        out_type=[
            jax.ShapeDtypeStruct((NC, NP, H), jnp.float32),
            jax.ShapeDtypeStruct((NC, NS, LANES), jnp.float32),
            jax.ShapeDtypeStruct((NC, LANES), jnp.float32),
        ],
        mesh=mesh,
        compiler_params=pltpu.CompilerParams(needs_layout_passes=False),
        scratch_types=[
            pltpu.VMEM((CH,), jnp.int32),
            pltpu.VMEM((CH,), jnp.int32),
            pltpu.VMEM((CH,), jnp.float32),
            pltpu.VMEM((CH,), jnp.float32),
            pltpu.VMEM((NCHUNK, CH), jnp.float32),
            pltpu.VMEM((CH, H), jnp.float32),
            pltpu.VMEM((RCP, H), jnp.float32),
            pltpu.VMEM((LANES,), jnp.float32),
            pltpu.VMEM((NS, LANES), jnp.float32),
            pltpu.VMEM_SHARED((N,), jnp.float32),
            pltpu.VMEM_SHARED((N,), jnp.float32),
            pltpu.VMEM_SHARED((NP, H), jnp.float32),
            pltpu.VMEM_SHARED((NS, LANES), jnp.float32),
            pltpu.SemaphoreType.DMA,
        ],
    )
    return fn(src1, dst1, asrc, adst, wh)


def _dense_out_body(p_ref, z_ref, m_ref, ow_ref, ob_ref, o_ref):
    m0 = m_ref[0, 0]
    m1 = m_ref[1, 0]
    mM = jnp.maximum(m0, m1)
    s0 = jnp.exp(m0 - mM)
    s1 = jnp.exp(m1 - mM)
    zz = s0 * jnp.sum(z_ref[0]) + s1 * jnp.sum(z_ref[1])
    hp = (p_ref[0] * s0 + p_ref[1] * s1) * (1.0 / zz)
    h2 = jnp.where(hp > 0.0, hp, jnp.exp(hp) - 1.0)
    o = jnp.dot(h2, ow_ref[...], preferred_element_type=jnp.float32)
    o = o + ob_ref[...][None, :]
    o = jnp.where(o > 0.0, o, jnp.exp(o) - 1.0)
    mx = jnp.max(o, axis=1, keepdims=True)
    lse = jnp.log(jnp.sum(jnp.exp(o - mx), axis=1, keepdims=True)) + mx
    o_ref[...] = o - lse


def _dense_out(part, zp, mp, out_W, out_b):
    blk = 2000
    grid = (N // blk,)
    return pl.pallas_call(
        _dense_out_body,
        grid=grid,
        in_specs=[
            pl.BlockSpec((NC, blk, H), lambda i: (0, i, 0)),
            pl.BlockSpec((NC, NS, LANES), lambda i: (0, 0, 0)),
            pl.BlockSpec((NC, LANES), lambda i: (0, 0)),
            pl.BlockSpec((H, C), lambda i: (0, 0)),
            pl.BlockSpec((C,), lambda i: (0,)),
        ],
        out_specs=pl.BlockSpec((blk, C), lambda i: (i, 0)),
        out_shape=jax.ShapeDtypeStruct((N, C), jnp.float32),
    )(part, zp, mp, out_W, out_b)


@jax.jit
def kernel(x, edge_index, emb_W, emb_b, W, a, out_W, out_b):
    a2 = jnp.stack([a[0, :H], a[0, H:]], axis=1)
    wh, alpha = _dense_in(x, emb_W, emb_b, W, a2)
    asrc = alpha[:, 0] + 0.0
    adst = alpha[:, 1] + 0.0
    part, zp, mp = _edge_kernel(edge_index[0], edge_index[1], asrc, adst, wh)
    return _dense_out(part, zp, mp, out_W, out_b)


# speedup vs baseline: 6.7133x; 1.3921x over previous
import functools

import jax
import jax.numpy as jnp
from jax import lax
from jax.experimental import pallas as pl
from jax.experimental.pallas import tpu as pltpu
from jax.experimental.pallas import tpu_sc as plsc

N = 10000
E = 320000
D = 128
H = 128
C = 128

NC = 2
NS = 16
LANES = 16
NW = NC * NS
EPT = E // NW
CH = 80
NCHUNK = EPT // CH
NGRP = CH // LANES
NP = 10240
RPT = NP // NS
RCP = 8
NEG = -9e15


def _dense_in_body(x_ref, ew_ref, eb_ref, w_ref, a2_ref, wh_ref, al_ref):
    h = jnp.dot(x_ref[...], ew_ref[...], preferred_element_type=jnp.float32)
    h = h + eb_ref[...][None, :]
    wh = jnp.dot(h, w_ref[...], preferred_element_type=jnp.float32)
    wh_ref[...] = wh
    al_ref[...] = jnp.dot(wh, a2_ref[...], preferred_element_type=jnp.float32)


def _dense_in(x, emb_W, emb_b, W, a2):
    blk = 2000
    grid = (N // blk,)
    return pl.pallas_call(
        _dense_in_body,
        grid=grid,
        in_specs=[
            pl.BlockSpec((blk, D), lambda i: (i, 0)),
            pl.BlockSpec((D, H), lambda i: (0, 0)),
            pl.BlockSpec((H,), lambda i: (0,)),
            pl.BlockSpec((H, H), lambda i: (0, 0)),
            pl.BlockSpec((H, 2), lambda i: (0, 0)),
        ],
        out_specs=[
            pl.BlockSpec((blk, H), lambda i: (i, 0)),
            pl.BlockSpec((blk, 2), lambda i: (i, 0)),
        ],
        out_shape=[
            jax.ShapeDtypeStruct((N, H), jnp.float32),
            jax.ShapeDtypeStruct((N, 2), jnp.float32),
        ],
    )(x, emb_W, emb_b, W, a2)


PB = 4


def _edge_body(src_hbm, dst_hbm, asrc_hbm, adst_hbm, wh_hbm,
               part_hbm, z_hbm, m_hbm,
               idxs_r, idxd_r, avals2, bvals2, rows2, zbuf_v,
               vec_v, stage_v, asrc_sh, adst_sh, acc_sh, stage_sh,
               isem0, isem1, isem2, isem3, gsem0, gsem1, ssem0, ssem1):
    cid = lax.axis_index("c")
    sid = lax.axis_index("s")
    gid = cid * NS + sid
    ebase = gid * EPT
    isems = [isem0, isem1, isem2, isem3]
    gsems = [gsem0, gsem1]
    ssems = [ssem0, ssem1]

    @pl.when(sid == 0)
    def _():
        pltpu.sync_copy(asrc_hbm, asrc_sh)
        pltpu.sync_copy(adst_hbm, adst_sh)

    zeros16 = jnp.zeros((LANES,), jnp.float32)
    for r in range(RCP):
        for f in range(H // LANES):
            zbuf_v[r, pl.ds(f * LANES, LANES)] = zeros16

    def _zero_step(j, carry):
        pltpu.sync_copy(zbuf_v, acc_sh.at[pl.ds(sid * RPT + j * RCP, RCP)])
        return carry
    lax.fori_loop(0, RPT // RCP, _zero_step, 0)
    plsc.subcore_barrier()

    def _masked(s):
        return jnp.where(s > 0.0, s, jnp.float32(NEG))

    def _issue_idx(c, b):
        base = ebase + c * CH
        d1 = pltpu.async_copy(src_hbm.at[pl.ds(base, CH)], idxs_r.at[b],
                              isems[b])
        d2 = pltpu.async_copy(dst_hbm.at[pl.ds(base, CH)], idxd_r.at[b],
                              isems[b])
        return (d1, d2)

    def _p1_blk(j, mx):
        c0 = j * PB
        di = [_issue_idx(c0 + b, b) for b in range(PB)]
        for b in range(PB):
            di[b][0].wait()
            di[b][1].wait()
            g1 = pltpu.async_copy(asrc_sh.at[idxs_r.at[b]], avals2.at[b],
                                  gsems[0])
            g1.wait()
            g2 = pltpu.async_copy(adst_sh.at[idxd_r.at[b]], bvals2.at[b],
                                  gsems[0])
            g2.wait()
            for g in range(NGRP):
                sl = pl.ds(g * LANES, LANES)
                mx = jnp.maximum(mx, _masked(avals2[b, sl] + bvals2[b, sl]))
        return mx

    mx = jnp.full((LANES,), jnp.float32(NEG))
    mx = lax.fori_loop(0, NCHUNK // PB, _p1_blk, mx)
    for c in range(PB * (NCHUNK // PB), NCHUNK):
        mx = _p1_blk(jnp.int32(c), mx) if False else mx
        d = _issue_idx(jnp.int32(c), 0)
        d[0].wait()
        d[1].wait()
        g1 = pltpu.async_copy(asrc_sh.at[idxs_r.at[0]], avals2.at[0],
                              gsems[0])
        g1.wait()
        g2 = pltpu.async_copy(adst_sh.at[idxd_r.at[0]], bvals2.at[0],
                              gsems[0])
        g2.wait()
        for g in range(NGRP):
            sl = pl.ds(g * LANES, LANES)
            mx = jnp.maximum(mx, _masked(avals2[0, sl] + bvals2[0, sl]))

    vec_v[...] = mx
    pltpu.sync_copy(vec_v, stage_sh.at[sid])
    plsc.subcore_barrier()
    pltpu.sync_copy(stage_sh, stage_v)
    mloc = jnp.full((LANES,), jnp.float32(NEG))
    for t in range(NS):
        mloc = jnp.maximum(mloc, stage_v[t, :])
    m_sc = jnp.max(mloc)

    def _p2_alpha_w(b, z):
        g1 = pltpu.async_copy(asrc_sh.at[idxs_r.at[b]], avals2.at[b],
                              gsems[0])
        g1.wait()
        g2 = pltpu.async_copy(adst_sh.at[idxd_r.at[b]], bvals2.at[b],
                              gsems[0])
        g2.wait()
        ws = []
        for g in range(NGRP):
            sl = pl.ds(g * LANES, LANES)
            w = jnp.exp(_masked(avals2[b, sl] + bvals2[b, sl]) - m_sc)
            z = z + w
            ws.append(w)
        return z, ws

    def _scale(b, ws):
        for g in range(NGRP):
            for k in range(LANES):
                wb = lax.broadcast(ws[g][k], (LANES,))
                r = g * LANES + k
                for f in range(H // LANES):
                    sl = pl.ds(f * LANES, LANES)
                    rows2[b, r, sl] = rows2[b, r, sl] * wb

    def _p2_pair(j, z):
        c0 = 2 * j
        i0 = _issue_idx(c0, 0)
        i1 = _issue_idx(c0 + 1, 1)
        i0[0].wait()
        i0[1].wait()
        z, ws0 = _p2_alpha_w(0, z)
        r0 = pltpu.async_copy(wh_hbm.at[idxs_r.at[0]], rows2.at[0], gsems[0])
        r0.wait()
        i1[0].wait()
        i1[1].wait()
        z, ws1 = _p2_alpha_w(1, z)
        r1 = pltpu.async_copy(wh_hbm.at[idxs_r.at[1]], rows2.at[1], gsems[1])
        _scale(0, ws0)
        s0 = pltpu.async_copy(rows2.at[0], acc_sh.at[idxd_r.at[0]], ssems[0],
                              add=True)
        r1.wait()
        _scale(1, ws1)
        s1 = pltpu.async_copy(rows2.at[1], acc_sh.at[idxd_r.at[1]], ssems[1],
                              add=True)
        s0.wait()
        s1.wait()
        return z

    z = jnp.zeros((LANES,), jnp.float32)
    z = lax.fori_loop(0, NCHUNK // 2, _p2_pair, z)
    iL = _issue_idx(jnp.int32(NCHUNK - 1), 0)
    iL[0].wait()
    iL[1].wait()
    z, wsL = _p2_alpha_w(0, z)
    rL = pltpu.async_copy(wh_hbm.at[idxs_r.at[0]], rows2.at[0], gsems[0])
    rL.wait()
    _scale(0, wsL)
    sL = pltpu.async_copy(rows2.at[0], acc_sh.at[idxd_r.at[0]], ssems[0],
                          add=True)
    sL.wait()

    vec_v[...] = z
    pltpu.sync_copy(vec_v, z_hbm.at[cid, sid])

    @pl.when(sid == 0)
    def _():
        vec_v[...] = lax.broadcast(m_sc, (LANES,))
        pltpu.sync_copy(vec_v, m_hbm.at[cid])

    plsc.subcore_barrier()
    pltpu.sync_copy(acc_sh.at[pl.ds(sid * RPT, RPT)],
                    part_hbm.at[cid, pl.ds(sid * RPT, RPT)])


def _edge_kernel(src1, dst1, asrc, adst, wh):
    mesh = plsc.VectorSubcoreMesh(core_axis_name="c", subcore_axis_name="s")
    fn = pl.kernel(
        _edge_body,
        out_type=[
            jax.ShapeDtypeStruct((NC, NP, H), jnp.float32),
            jax.ShapeDtypeStruct((NC, NS, LANES), jnp.float32),
            jax.ShapeDtypeStruct((NC, LANES), jnp.float32),
        ],
        mesh=mesh,
        compiler_params=pltpu.CompilerParams(needs_layout_passes=False),
        scratch_types=[
            pltpu.VMEM((PB, CH), jnp.int32),
            pltpu.VMEM((PB, CH), jnp.int32),
            pltpu.VMEM((PB, CH), jnp.float32),
            pltpu.VMEM((PB, CH), jnp.float32),
            pltpu.VMEM((2, CH, H), jnp.float32),
            pltpu.VMEM((RCP, H), jnp.float32),
            pltpu.VMEM((LANES,), jnp.float32),
            pltpu.VMEM((NS, LANES), jnp.float32),
            pltpu.VMEM_SHARED((N,), jnp.float32),
            pltpu.VMEM_SHARED((N,), jnp.float32),
            pltpu.VMEM_SHARED((NP, H), jnp.float32),
            pltpu.VMEM_SHARED((NS, LANES), jnp.float32),
            pltpu.SemaphoreType.DMA,
            pltpu.SemaphoreType.DMA,
            pltpu.SemaphoreType.DMA,
            pltpu.SemaphoreType.DMA,
            pltpu.SemaphoreType.DMA,
            pltpu.SemaphoreType.DMA,
            pltpu.SemaphoreType.DMA,
            pltpu.SemaphoreType.DMA,
        ],
    )
    return fn(src1, dst1, asrc, adst, wh)


def _dense_out_body(p_ref, z_ref, m_ref, ow_ref, ob_ref, o_ref):
    m0 = m_ref[0, 0]
    m1 = m_ref[1, 0]
    mM = jnp.maximum(m0, m1)
    s0 = jnp.exp(m0 - mM)
    s1 = jnp.exp(m1 - mM)
    zz = s0 * jnp.sum(z_ref[0]) + s1 * jnp.sum(z_ref[1])
    hp = (p_ref[0] * s0 + p_ref[1] * s1) * (1.0 / zz)
    h2 = jnp.where(hp > 0.0, hp, jnp.exp(hp) - 1.0)
    o = jnp.dot(h2, ow_ref[...], preferred_element_type=jnp.float32)
    o = o + ob_ref[...][None, :]
    o = jnp.where(o > 0.0, o, jnp.exp(o) - 1.0)
    mx = jnp.max(o, axis=1, keepdims=True)
    lse = jnp.log(jnp.sum(jnp.exp(o - mx), axis=1, keepdims=True)) + mx
    o_ref[...] = o - lse


def _dense_out(part, zp, mp, out_W, out_b):
    blk = 2000
    grid = (N // blk,)
    return pl.pallas_call(
        _dense_out_body,
        grid=grid,
        in_specs=[
            pl.BlockSpec((NC, blk, H), lambda i: (0, i, 0)),
            pl.BlockSpec((NC, NS, LANES), lambda i: (0, 0, 0)),
            pl.BlockSpec((NC, LANES), lambda i: (0, 0)),
            pl.BlockSpec((H, C), lambda i: (0, 0)),
            pl.BlockSpec((C,), lambda i: (0,)),
        ],
        out_specs=pl.BlockSpec((blk, C), lambda i: (i, 0)),
        out_shape=jax.ShapeDtypeStruct((N, C), jnp.float32),
    )(part, zp, mp, out_W, out_b)


@jax.jit
def kernel(x, edge_index, emb_W, emb_b, W, a, out_W, out_b):
    a2 = jnp.stack([a[0, :H], a[0, H:]], axis=1)
    wh, alpha = _dense_in(x, emb_W, emb_b, W, a2)
    asrc = alpha[:, 0] + 0.0
    adst = alpha[:, 1] + 0.0
    part, zp, mp = _edge_kernel(edge_index[0], edge_index[1], asrc, adst, wh)
    return _dense_out(part, zp, mp, out_W, out_b)
